# Initial kernel scaffold; baseline (speedup 1.0000x reference)
import jax
import jax.numpy as jnp
from jax.experimental import pallas as pl


def kernel(x, edge_index, edge_attr, batch, t1, We1, m1W1, m1W2, t2, Ws2, Wd2, We2, m2W1, m2W2, t3, Ws3, Wd3, We3, m3W1, m3W2, fc1_W, fc1_b, fc2_W, fc2_b):
    raise NotImplementedError("write your pallas kernel here")


# speedup vs baseline: 3.2918x; 3.2918x over previous
import math
import jax
import jax.numpy as jnp
from jax import lax
from jax.experimental import pallas as pl
from jax.experimental.pallas import tpu as pltpu
from jax.experimental.pallas import tpu_sc as plsc

N = 10000
E = 320000
G = 16
NP = 10240
NACC = 10368
TRASH = NP
K = 128
EPAD = 323584
EPT1 = EPAD // 16
NCH1 = EPT1 // K
EPT2 = EPAD // 32
NCH2 = EPT2 // K
RPT = NACC // 16
ZB = 32
BLK = 512
NBLK = NP // BLK
EBLK = EPAD // BLK
BN_INV = 1.0 / math.sqrt(1.0 + 1e-5)
MSG_EPS = 1e-7
F32 = jnp.float32


def _sc_body(D, split):
  Dh = D // 2 if split else D
  W = Dh // 16
  NCH = NCH1 if split else NCH2

  def body(table, eproj, src, dst, par, out0, out1,
           acc, srcb, dstb, erows, pwb, parb, gsem):
    c = lax.axis_index("c")
    s = lax.axis_index("s")

    def zz(r, _):
      for k in range(8):
        pwb[r, pl.ds(16 * k, 16)] = jnp.zeros((16,), F32)
      return 0
    lax.fori_loop(0, K, zz, 0)
    r0 = s * RPT
    done = 0
    while done < RPT:
      sz = min(K, RPT - done)
      pltpu.sync_copy(pwb.at[pl.ds(0, sz)], acc.at[pl.ds(r0 + done, sz)])
      done += sz
    pltpu.sync_copy(par, parb)
    plsc.subcore_barrier()

    tv = parb[0, :]
    mh = parb[1, :]
    me = parb[2, :]
    cv = tv * (jnp.maximum(mh + me, 0.0) + MSG_EPS)

    if split:
      eb = s * EPT1
      xoff = c * Dh
    else:
      eb = (c * 16 + s) * EPT2
      xoff = 0

    def chunk(j, _):
      e0 = eb + j * K
      pltpu.sync_copy(src.at[pl.ds(e0, K)], srcb)
      pltpu.sync_copy(dst.at[pl.ds(e0, K)], dstb.at[0])
      if split:
        pltpu.sync_copy(eproj.at[c, pl.ds(e0, K)], erows)
      else:
        pltpu.sync_copy(eproj.at[pl.ds(e0, K)], erows)
      pltpu.async_copy(table.at[srcb], pwb, gsem).wait()

      def ce(i, _):
        for k in range(W):
          xv = pwb[i, pl.ds(xoff + 16 * k, 16)]
          ev = erows[i, pl.ds(16 * k, 16)]
          v = jnp.maximum(xv + ev, 0.0) + MSG_EPS
          p = jnp.exp(tv * v - cv)
          pwb[i, pl.ds(16 * k, 16)] = p
          pwb[i, pl.ds(Dh + 16 * k, 16)] = v * p
        return 0
      lax.fori_loop(0, K, ce, 0)
      pltpu.sync_copy(pwb, acc.at[dstb.at[0]], add=True)
      return 0
    lax.fori_loop(0, NCH, chunk, 0)
    plsc.subcore_barrier()

    @pl.when(c == 0)
    def _():
      pltpu.sync_copy(acc.at[pl.ds(r0, RPT)], out0.at[pl.ds(r0, RPT)])

    @pl.when(c == 1)
    def _():
      pltpu.sync_copy(acc.at[pl.ds(r0, RPT)], out1.at[pl.ds(r0, RPT)])
  return body


def _sc_pass(table, eproj, src, dst, par, D, split):
  Dh = D // 2 if split else D
  body = _sc_body(D, split)
  out_t = [jax.ShapeDtypeStruct((NACC, 128), F32)] * 2
  f = pl.kernel(
      body,
      out_type=out_t,
      mesh=plsc.VectorSubcoreMesh(core_axis_name="c", subcore_axis_name="s"),
      scratch_types=[
          pltpu.VMEM_SHARED((NACC, 128), F32),
          pltpu.VMEM((K,), jnp.int32),
          pltpu.VMEM((1, K), jnp.int32),
          pltpu.VMEM((K, Dh), F32),
          pltpu.VMEM((K, 128), F32),
          pltpu.VMEM((4, 16), F32),
          pltpu.SemaphoreType.DMA,
      ],
  )
  return f(table, eproj, src, dst, par)


def _maxinit_update(ref, val, first):
  @pl.when(first)
  def _():
    ref[...] = jnp.full((8, 128), -1e30, F32)
  ref[...] = jnp.maximum(ref[...], val)


def _tc_max(xp):
  def body(xr, mx):
    i = pl.program_id(0)
    _maxinit_update(mx, jnp.max(xr[...]), i == 0)
  return pl.pallas_call(
      body,
      grid=(NBLK,),
      in_specs=[pl.BlockSpec((BLK, 128), lambda i: (i, 0))],
      out_specs=pl.BlockSpec((8, 128), lambda i: (0, 0)),
      out_shape=jax.ShapeDtypeStruct((8, 128), F32),
  )(xp)


def _tc_eproj(eap, We1, We2, We3):
  def body(ea, w1, w2, w3, e1, e2, e3, m1, m2, m3):
    i = pl.program_id(0)
    c = pl.program_id(1)
    first = (c == 0) & (i == 0)
    a = ea[...]
    b1 = jnp.dot(a, w1[...], preferred_element_type=F32)

    @pl.when(c == 0)
    def _():
      e1[...] = b1[:, :64]
      e2[...] = jnp.dot(a, w2[...], preferred_element_type=F32)
      e3[...] = jnp.dot(a, w3[...], preferred_element_type=F32)

    @pl.when(c == 1)
    def _():
      e1[...] = b1[:, 64:]

    _maxinit_update(m1, jnp.max(b1), first)
    _maxinit_update(m2, jnp.max(e2[...]), first)
    _maxinit_update(m3, jnp.max(e3[...]), first)

  mt = jax.ShapeDtypeStruct((8, 128), F32)
  return pl.pallas_call(
      body,
      grid=(EBLK, 2),
      in_specs=[
          pl.BlockSpec((BLK, 16), lambda i, c: (i, 0)),
          pl.BlockSpec((16, 128), lambda i, c: (0, 0)),
          pl.BlockSpec((16, 64), lambda i, c: (0, 0)),
          pl.BlockSpec((16, 32), lambda i, c: (0, 0)),
      ],
      out_specs=[
          pl.BlockSpec((None, BLK, 64), lambda i, c: (c, i, 0)),
          pl.BlockSpec((BLK, 64), lambda i, c: (i, 0)),
          pl.BlockSpec((BLK, 32), lambda i, c: (i, 0)),
          pl.BlockSpec((8, 128), lambda i, c: (0, 0)),
          pl.BlockSpec((8, 128), lambda i, c: (0, 0)),
          pl.BlockSpec((8, 128), lambda i, c: (0, 0)),
      ],
      out_shape=[
          jax.ShapeDtypeStruct((2, EPAD, 64), F32),
          jax.ShapeDtypeStruct((EPAD, 64), F32),
          jax.ShapeDtypeStruct((EPAD, 32), F32),
          mt, mt, mt,
      ],
  )(eap, We1, We2, We3)


def _tc_post(pw0, pw1, xadd, mW1, mW2, Ws, Wd, D, split):
  DM = mW1.shape[1]
  DO = mW2.shape[1]
  DS = Ws.shape[1]

  def body(p0r, p1r, xr, w1, w2, ws, wd, hs, xd, mh):
    i = pl.program_id(0)
    p0 = p0r[...]
    p1 = p1r[...]
    xa = xr[...][:, :D]
    if split:
      Dh = D // 2
      a0 = p0[:, 64:64 + Dh] / (p0[:, :Dh] + 1e-16) + xa[:, :Dh]
      a1 = p1[:, 64:64 + Dh] / (p1[:, :Dh] + 1e-16) + xa[:, Dh:]
      h = (jnp.dot(a0, w1[...][:Dh, :], preferred_element_type=F32)
           + jnp.dot(a1, w1[...][Dh:, :], preferred_element_type=F32))
    else:
      S = p0[:, :D] + p1[:, :D]
      Wv = p0[:, D:2 * D] + p1[:, D:2 * D]
      a = Wv / (S + 1e-16) + xa
      h = jnp.dot(a, w1[...], preferred_element_type=F32)
    h = jnp.maximum(h * BN_INV, 0.0)
    h2 = jnp.maximum(jnp.dot(h, w2[...], preferred_element_type=F32), 0.0)
    hsb = jnp.dot(h2, ws[...], preferred_element_type=F32)
    hs[...] = jnp.concatenate(
        [hsb, jnp.zeros((BLK, 128 - DS), F32)], axis=1)
    xd[...] = jnp.dot(h2, wd[...], preferred_element_type=F32)
    _maxinit_update(mh, jnp.max(hsb), i == 0)

  return pl.pallas_call(
      body,
      grid=(NBLK,),
      in_specs=[
          pl.BlockSpec((BLK, 128), lambda i: (i, 0)),
          pl.BlockSpec((BLK, 128), lambda i: (i, 0)),
          pl.BlockSpec((BLK, xadd.shape[1]), lambda i: (i, 0)),
          pl.BlockSpec((D, DM), lambda i: (0, 0)),
          pl.BlockSpec((DM, DO), lambda i: (0, 0)),
          pl.BlockSpec((DO, DS), lambda i: (0, 0)),
          pl.BlockSpec((DO, DS), lambda i: (0, 0)),
      ],
      out_specs=[
          pl.BlockSpec((BLK, 128), lambda i: (i, 0)),
          pl.BlockSpec((BLK, DS), lambda i: (i, 0)),
          pl.BlockSpec((8, 128), lambda i: (0, 0)),
      ],
      out_shape=[
          jax.ShapeDtypeStruct((NP, 128), F32),
          jax.ShapeDtypeStruct((NP, DS), F32),
          jax.ShapeDtypeStruct((8, 128), F32),
      ],
  )(pw0, pw1, xadd, mW1, mW2, Ws, Wd)


def _tc_head(pw0, pw1, xd3, mW1, mW2, b2d, fw1, fb1, fw2, fb2):
  def body(p0r, p1r, xr, w1, w2, bb, f1, b1, f2, b2, out, pooled, counts):
    i = pl.program_id(0)
    p0 = p0r[...]
    p1 = p1r[...]
    S = p0[:, :32] + p1[:, :32]
    Wv = p0[:, 32:64] + p1[:, 32:64]
    a = Wv / (S + 1e-16) + xr[...]
    h = jnp.dot(a, w1[...], preferred_element_type=F32)
    h = jnp.maximum(h * BN_INV, 0.0)
    h3 = jnp.maximum(jnp.dot(h, w2[...], preferred_element_type=F32), 0.0)

    oh = (lax.broadcasted_iota(jnp.int32, (G, BLK), 0) == bb[...]).astype(F32)

    @pl.when(i == 0)
    def _():
      pooled[...] = jnp.zeros((G, 32), F32)
      counts[...] = jnp.zeros((G, 128), F32)

    pooled[...] += jnp.dot(oh, h3, preferred_element_type=F32)
    cnt = jnp.sum(oh, axis=1, keepdims=True)
    counts[...] += jnp.broadcast_to(cnt, (G, 128))

    @pl.when(i == NBLK - 1)
    def _():
      pm = pooled[...] / jnp.maximum(counts[:, 0:1], 1.0)
      z = jnp.maximum(jnp.dot(pm, f1[...], preferred_element_type=F32)
                      + b1[...], 0.0)
      o = jnp.dot(z, f2[...], preferred_element_type=F32) + b2[...]
      out[...] = jax.nn.sigmoid(o)

  return pl.pallas_call(
      body,
      grid=(NBLK,),
      in_specs=[
          pl.BlockSpec((BLK, 128), lambda i: (i, 0)),
          pl.BlockSpec((BLK, 128), lambda i: (i, 0)),
          pl.BlockSpec((BLK, 32), lambda i: (i, 0)),
          pl.BlockSpec((32, 64), lambda i: (0, 0)),
          pl.BlockSpec((64, 32), lambda i: (0, 0)),
          pl.BlockSpec((1, BLK), lambda i: (0, i)),
          pl.BlockSpec((32, 64), lambda i: (0, 0)),
          pl.BlockSpec((1, 64), lambda i: (0, 0)),
          pl.BlockSpec((64, 1), lambda i: (0, 0)),
          pl.BlockSpec((1, 1), lambda i: (0, 0)),
      ],
      out_specs=pl.BlockSpec((G, 1), lambda i: (0, 0)),
      out_shape=jax.ShapeDtypeStruct((G, 1), F32),
      scratch_shapes=[
          pltpu.VMEM((G, 32), F32),
          pltpu.VMEM((G, 128), F32),
      ],
  )(pw0, pw1, xd3, mW1, mW2, b2d, fw1, fb1, fw2, fb2)


def _par(t, mhbuf, mebuf):
  return jnp.stack([
      jnp.full((16,), t, F32),
      mhbuf[0, :16],
      mebuf[0, :16],
      jnp.zeros((16,), F32),
  ])


def kernel(x, edge_index, edge_attr, batch, t1, We1, m1W1, m1W2, t2, Ws2, Wd2,
           We2, m2W1, m2W2, t3, Ws3, Wd3, We3, m3W1, m3W2, fc1_W, fc1_b,
           fc2_W, fc2_b):
  srcp = jnp.pad(edge_index[0], (0, EPAD - E))
  dstp = jnp.pad(edge_index[1], (0, EPAD - E), constant_values=TRASH)
  eap = jnp.pad(edge_attr, ((0, EPAD - E), (0, 0)))
  xp = jnp.pad(x, ((0, NP - N), (0, 0)))
  b2d = jnp.pad(batch, (0, NP - N), constant_values=G).reshape(1, NP)

  e1, e2, e3, me1, me2, me3 = _tc_eproj(eap, We1, We2, We3)
  mx = _tc_max(xp)

  pw10, pw11 = _sc_pass(xp, e1, srcp, dstp, _par(t1, mx, me1), 128, True)
  hs2, xd2, mh2 = _tc_post(pw10[:NP], pw11[:NP], xp, m1W1, m1W2, Ws2, Wd2,
                           128, True)

  pw20, pw21 = _sc_pass(hs2, e2, srcp, dstp, _par(t2, mh2, me2), 64, False)
  hs3, xd3, mh3 = _tc_post(pw20[:NP], pw21[:NP], xd2, m2W1, m2W2, Ws3, Wd3,
                           64, False)

  pw30, pw31 = _sc_pass(hs3, e3, srcp, dstp, _par(t3, mh3, me3), 32, False)
  return _tc_head(pw30[:NP], pw31[:NP], xd3, m3W1, m3W2, b2d,
                  fc1_W, fc1_b.reshape(1, 64), fc2_W, fc2_b.reshape(1, 1))


# speedup vs baseline: 3.6278x; 1.1021x over previous
import math
import jax
import jax.numpy as jnp
from jax import lax
from jax.experimental import pallas as pl
from jax.experimental.pallas import tpu as pltpu
from jax.experimental.pallas import tpu_sc as plsc

N = 10000
E = 320000
G = 16
NP = 10240
NACC = 10368
TRASH = NP
K = 64
EPAD = 323584
EPT1 = EPAD // 16
NCH1 = EPT1 // K
EPT2 = EPAD // 32
NCH2 = EPT2 // K
RPT = NACC // 16
ZB = 32
BLK = 512
NBLK = NP // BLK
EBLK = EPAD // BLK
BN_INV = 1.0 / math.sqrt(1.0 + 1e-5)
MSG_EPS = 1e-7
F32 = jnp.float32


def _sc_body(D, split):
  Dh = D // 2 if split else D
  W = Dh // 16
  NCH = NCH1 if split else NCH2

  def body(table, eproj, src, dst, par, out0, out1,
           acc, srcb, dstb, erows, xrows, pwb, parb, gsem0, gsem1):
    c = lax.axis_index("c")
    s = lax.axis_index("s")

    def zz(r, _):
      for k in range(8):
        pwb[r, pl.ds(16 * k, 16)] = jnp.zeros((16,), F32)
      return 0
    lax.fori_loop(0, K, zz, 0)
    r0 = s * RPT
    done = 0
    while done < RPT:
      sz = min(K, RPT - done)
      pltpu.sync_copy(pwb.at[pl.ds(0, sz)], acc.at[pl.ds(r0 + done, sz)])
      done += sz
    pltpu.sync_copy(par, parb)
    plsc.subcore_barrier()

    tv = parb[0, :]
    mh = parb[1, :]
    me = parb[2, :]
    cv = tv * (jnp.maximum(mh + me, 0.0) + MSG_EPS)

    if split:
      eb = s * EPT1
      xoff = c * Dh
    else:
      eb = (c * 16 + s) * EPT2
      xoff = 0

    sems = (gsem0, gsem1)

    def load_small(j, b):
      e0 = eb + j * K
      pltpu.sync_copy(src.at[pl.ds(e0, K)], srcb.at[b])
      pltpu.sync_copy(dst.at[pl.ds(e0, K)], dstb.at[b])
      if split:
        pltpu.sync_copy(eproj.at[c, pl.ds(e0, K)], erows.at[b])
      else:
        pltpu.sync_copy(eproj.at[pl.ds(e0, K)], erows.at[b])

    def start_gather(b):
      pltpu.async_copy(table.at[srcb.at[b]], xrows.at[b], sems[b])

    def consume(b):
      pltpu.make_async_copy(table.at[srcb.at[b]], xrows.at[b], sems[b]).wait()

      def ce(i2, _):
        for u in range(2):
          i = i2 * 2 + u
          for k in range(W):
            xv = xrows[b, i, pl.ds(xoff + 16 * k, 16)]
            ev = erows[b, i, pl.ds(16 * k, 16)]
            v = jnp.maximum(xv + ev, 0.0) + MSG_EPS
            p = jnp.exp(tv * v - cv)
            pwb[i, pl.ds(16 * k, 16)] = p
            pwb[i, pl.ds(Dh + 16 * k, 16)] = v * p
        return 0
      lax.fori_loop(0, K // 2, ce, 0)
      pltpu.sync_copy(pwb, acc.at[dstb.at[b]], add=True)

    load_small(0, 0)
    start_gather(0)

    def loop(jj, _):
      j0 = 2 * jj
      load_small(j0 + 1, 1)
      start_gather(1)
      consume(0)
      load_small(j0 + 2, 0)
      start_gather(0)
      consume(1)
      return 0
    lax.fori_loop(0, NCH // 2 - 1, loop, 0)
    load_small(NCH - 1, 1)
    start_gather(1)
    consume(0)
    consume(1)
    plsc.subcore_barrier()

    @pl.when(c == 0)
    def _():
      pltpu.sync_copy(acc.at[pl.ds(r0, RPT)], out0.at[pl.ds(r0, RPT)])

    @pl.when(c == 1)
    def _():
      pltpu.sync_copy(acc.at[pl.ds(r0, RPT)], out1.at[pl.ds(r0, RPT)])
  return body


def _sc_pass(table, eproj, src, dst, par, D, split):
  Dh = D // 2 if split else D
  body = _sc_body(D, split)
  out_t = [jax.ShapeDtypeStruct((NACC, 128), F32)] * 2
  f = pl.kernel(
      body,
      out_type=out_t,
      mesh=plsc.VectorSubcoreMesh(core_axis_name="c", subcore_axis_name="s"),
      scratch_types=[
          pltpu.VMEM_SHARED((NACC, 128), F32),
          pltpu.VMEM((2, K), jnp.int32),
          pltpu.VMEM((2, K), jnp.int32),
          pltpu.VMEM((2, K, Dh), F32),
          pltpu.VMEM((2, K, 128), F32),
          pltpu.VMEM((K, 128), F32),
          pltpu.VMEM((4, 16), F32),
          pltpu.SemaphoreType.DMA,
          pltpu.SemaphoreType.DMA,
      ],
  )
  return f(table, eproj, src, dst, par)


def _maxinit_update(ref, val, first):
  @pl.when(first)
  def _():
    ref[...] = jnp.full((8, 128), -1e30, F32)
  ref[...] = jnp.maximum(ref[...], val)


def _tc_max(xp):
  def body(xr, mx):
    i = pl.program_id(0)
    _maxinit_update(mx, jnp.max(xr[...]), i == 0)
  return pl.pallas_call(
      body,
      grid=(NBLK,),
      in_specs=[pl.BlockSpec((BLK, 128), lambda i: (i, 0))],
      out_specs=pl.BlockSpec((8, 128), lambda i: (0, 0)),
      out_shape=jax.ShapeDtypeStruct((8, 128), F32),
  )(xp)


def _tc_eproj(eap, We1, We2, We3):
  def body(ea, w1, w2, w3, e1, e2, e3, m1, m2, m3):
    i = pl.program_id(0)
    c = pl.program_id(1)
    first = (c == 0) & (i == 0)
    a = ea[...]
    b1 = jnp.dot(a, w1[...], preferred_element_type=F32)

    @pl.when(c == 0)
    def _():
      e1[...] = b1[:, :64]
      e2[...] = jnp.dot(a, w2[...], preferred_element_type=F32)
      e3[...] = jnp.dot(a, w3[...], preferred_element_type=F32)

    @pl.when(c == 1)
    def _():
      e1[...] = b1[:, 64:]

    _maxinit_update(m1, jnp.max(b1), first)
    _maxinit_update(m2, jnp.max(e2[...]), first)
    _maxinit_update(m3, jnp.max(e3[...]), first)

  mt = jax.ShapeDtypeStruct((8, 128), F32)
  return pl.pallas_call(
      body,
      grid=(EBLK, 2),
      in_specs=[
          pl.BlockSpec((BLK, 16), lambda i, c: (i, 0)),
          pl.BlockSpec((16, 128), lambda i, c: (0, 0)),
          pl.BlockSpec((16, 64), lambda i, c: (0, 0)),
          pl.BlockSpec((16, 32), lambda i, c: (0, 0)),
      ],
      out_specs=[
          pl.BlockSpec((None, BLK, 64), lambda i, c: (c, i, 0)),
          pl.BlockSpec((BLK, 64), lambda i, c: (i, 0)),
          pl.BlockSpec((BLK, 32), lambda i, c: (i, 0)),
          pl.BlockSpec((8, 128), lambda i, c: (0, 0)),
          pl.BlockSpec((8, 128), lambda i, c: (0, 0)),
          pl.BlockSpec((8, 128), lambda i, c: (0, 0)),
      ],
      out_shape=[
          jax.ShapeDtypeStruct((2, EPAD, 64), F32),
          jax.ShapeDtypeStruct((EPAD, 64), F32),
          jax.ShapeDtypeStruct((EPAD, 32), F32),
          mt, mt, mt,
      ],
  )(eap, We1, We2, We3)


def _tc_post(pw0, pw1, xadd, mW1, mW2, Ws, Wd, D, split):
  DM = mW1.shape[1]
  DO = mW2.shape[1]
  DS = Ws.shape[1]

  def body(p0r, p1r, xr, w1, w2, ws, wd, hs, xd, mh):
    i = pl.program_id(0)
    p0 = p0r[...]
    p1 = p1r[...]
    xa = xr[...][:, :D]
    if split:
      Dh = D // 2
      a0 = p0[:, 64:64 + Dh] / (p0[:, :Dh] + 1e-16) + xa[:, :Dh]
      a1 = p1[:, 64:64 + Dh] / (p1[:, :Dh] + 1e-16) + xa[:, Dh:]
      h = (jnp.dot(a0, w1[...][:Dh, :], preferred_element_type=F32)
           + jnp.dot(a1, w1[...][Dh:, :], preferred_element_type=F32))
    else:
      S = p0[:, :D] + p1[:, :D]
      Wv = p0[:, D:2 * D] + p1[:, D:2 * D]
      a = Wv / (S + 1e-16) + xa
      h = jnp.dot(a, w1[...], preferred_element_type=F32)
    h = jnp.maximum(h * BN_INV, 0.0)
    h2 = jnp.maximum(jnp.dot(h, w2[...], preferred_element_type=F32), 0.0)
    hsb = jnp.dot(h2, ws[...], preferred_element_type=F32)
    hs[...] = jnp.concatenate(
        [hsb, jnp.zeros((BLK, 128 - DS), F32)], axis=1)
    xd[...] = jnp.dot(h2, wd[...], preferred_element_type=F32)
    _maxinit_update(mh, jnp.max(hsb), i == 0)

  return pl.pallas_call(
      body,
      grid=(NBLK,),
      in_specs=[
          pl.BlockSpec((BLK, 128), lambda i: (i, 0)),
          pl.BlockSpec((BLK, 128), lambda i: (i, 0)),
          pl.BlockSpec((BLK, xadd.shape[1]), lambda i: (i, 0)),
          pl.BlockSpec((D, DM), lambda i: (0, 0)),
          pl.BlockSpec((DM, DO), lambda i: (0, 0)),
          pl.BlockSpec((DO, DS), lambda i: (0, 0)),
          pl.BlockSpec((DO, DS), lambda i: (0, 0)),
      ],
      out_specs=[
          pl.BlockSpec((BLK, 128), lambda i: (i, 0)),
          pl.BlockSpec((BLK, DS), lambda i: (i, 0)),
          pl.BlockSpec((8, 128), lambda i: (0, 0)),
      ],
      out_shape=[
          jax.ShapeDtypeStruct((NP, 128), F32),
          jax.ShapeDtypeStruct((NP, DS), F32),
          jax.ShapeDtypeStruct((8, 128), F32),
      ],
  )(pw0, pw1, xadd, mW1, mW2, Ws, Wd)


def _tc_head(pw0, pw1, xd3, mW1, mW2, b2d, fw1, fb1, fw2, fb2):
  def body(p0r, p1r, xr, w1, w2, bb, f1, b1, f2, b2, out, pooled, counts):
    i = pl.program_id(0)
    p0 = p0r[...]
    p1 = p1r[...]
    S = p0[:, :32] + p1[:, :32]
    Wv = p0[:, 32:64] + p1[:, 32:64]
    a = Wv / (S + 1e-16) + xr[...]
    h = jnp.dot(a, w1[...], preferred_element_type=F32)
    h = jnp.maximum(h * BN_INV, 0.0)
    h3 = jnp.maximum(jnp.dot(h, w2[...], preferred_element_type=F32), 0.0)

    oh = (lax.broadcasted_iota(jnp.int32, (G, BLK), 0) == bb[...]).astype(F32)

    @pl.when(i == 0)
    def _():
      pooled[...] = jnp.zeros((G, 32), F32)
      counts[...] = jnp.zeros((G, 128), F32)

    pooled[...] += jnp.dot(oh, h3, preferred_element_type=F32)
    cnt = jnp.sum(oh, axis=1, keepdims=True)
    counts[...] += jnp.broadcast_to(cnt, (G, 128))

    @pl.when(i == NBLK - 1)
    def _():
      pm = pooled[...] / jnp.maximum(counts[:, 0:1], 1.0)
      z = jnp.maximum(jnp.dot(pm, f1[...], preferred_element_type=F32)
                      + b1[...], 0.0)
      o = jnp.dot(z, f2[...], preferred_element_type=F32) + b2[...]
      out[...] = jax.nn.sigmoid(o)

  return pl.pallas_call(
      body,
      grid=(NBLK,),
      in_specs=[
          pl.BlockSpec((BLK, 128), lambda i: (i, 0)),
          pl.BlockSpec((BLK, 128), lambda i: (i, 0)),
          pl.BlockSpec((BLK, 32), lambda i: (i, 0)),
          pl.BlockSpec((32, 64), lambda i: (0, 0)),
          pl.BlockSpec((64, 32), lambda i: (0, 0)),
          pl.BlockSpec((1, BLK), lambda i: (0, i)),
          pl.BlockSpec((32, 64), lambda i: (0, 0)),
          pl.BlockSpec((1, 64), lambda i: (0, 0)),
          pl.BlockSpec((64, 1), lambda i: (0, 0)),
          pl.BlockSpec((1, 1), lambda i: (0, 0)),
      ],
      out_specs=pl.BlockSpec((G, 1), lambda i: (0, 0)),
      out_shape=jax.ShapeDtypeStruct((G, 1), F32),
      scratch_shapes=[
          pltpu.VMEM((G, 32), F32),
          pltpu.VMEM((G, 128), F32),
      ],
  )(pw0, pw1, xd3, mW1, mW2, b2d, fw1, fb1, fw2, fb2)


def _par(t, mhbuf, mebuf):
  return jnp.stack([
      jnp.full((16,), t, F32),
      mhbuf[0, :16],
      mebuf[0, :16],
      jnp.zeros((16,), F32),
  ])


def kernel(x, edge_index, edge_attr, batch, t1, We1, m1W1, m1W2, t2, Ws2, Wd2,
           We2, m2W1, m2W2, t3, Ws3, Wd3, We3, m3W1, m3W2, fc1_W, fc1_b,
           fc2_W, fc2_b):
  srcp = jnp.pad(edge_index[0], (0, EPAD - E))
  dstp = jnp.pad(edge_index[1], (0, EPAD - E), constant_values=TRASH)
  eap = jnp.pad(edge_attr, ((0, EPAD - E), (0, 0)))
  xp = jnp.pad(x, ((0, NP - N), (0, 0)))
  b2d = jnp.pad(batch, (0, NP - N), constant_values=G).reshape(1, NP)

  e1, e2, e3, me1, me2, me3 = _tc_eproj(eap, We1, We2, We3)
  mx = _tc_max(xp)

  pw10, pw11 = _sc_pass(xp, e1, srcp, dstp, _par(t1, mx, me1), 128, True)
  hs2, xd2, mh2 = _tc_post(pw10[:NP], pw11[:NP], xp, m1W1, m1W2, Ws2, Wd2,
                           128, True)

  pw20, pw21 = _sc_pass(hs2, e2, srcp, dstp, _par(t2, mh2, me2), 64, False)
  hs3, xd3, mh3 = _tc_post(pw20[:NP], pw21[:NP], xd2, m2W1, m2W2, Ws3, Wd3,
                           64, False)

  pw30, pw31 = _sc_pass(hs3, e3, srcp, dstp, _par(t3, mh3, me3), 32, False)
  return _tc_head(pw30[:NP], pw31[:NP], xd3, m3W1, m3W2, b2d,
                  fc1_W, fc1_b.reshape(1, 64), fc2_W, fc2_b.reshape(1, 1))
